# baseline (speedup 1.0000x reference)
import functools

import jax
import jax.numpy as jnp
from jax import lax
from jax.experimental import pallas as pl
from jax.experimental.pallas import tpu as pltpu
from jax.experimental.pallas import tpu_sc as plsc

_BATCH = 16384
_FIELDS = 100
_FPAD = 104
_DIM = 64
_NC = 2
_NS = 16
_NW = _NC * _NS
_BPW = _BATCH // _NW


def _bip_sc(x_pad, emb):
  mesh = plsc.VectorSubcoreMesh(core_axis_name="c", subcore_axis_name="s")

  nbuf = 4

  @functools.partial(
      pl.kernel,
      out_type=jax.ShapeDtypeStruct((_BATCH, _DIM), jnp.float32),
      mesh=mesh,
      scratch_types=(
          [pltpu.VMEM((_BPW, _FPAD), jnp.int32)]
          + [pltpu.VMEM((_FPAD, _DIM), jnp.float32)
             for _ in range(nbuf)]
          + [pltpu.VMEM((_BPW, _DIM), jnp.float32)]
          + [pltpu.SemaphoreType.DMA for _ in range(nbuf)]
      ),
      compiler_params=pltpu.CompilerParams(use_tc_tiling_on_sc=False),
  )
  def k(x_hbm, emb_hbm, h_hbm, idx_v, *rest):
    rows_bufs = rest[:nbuf]
    out_v = rest[nbuf]
    sems = rest[nbuf + 1:]
    wid = lax.axis_index("s") * _NC + lax.axis_index("c")
    base = wid * _BPW
    pltpu.sync_copy(x_hbm.at[pl.ds(base, _BPW)], idx_v)

    def start(i, b):
      pltpu.make_async_copy(emb_hbm.at[idx_v.at[i]], rows_bufs[b],
                            sems[b]).start()

    def wait(b):
      pltpu.make_async_copy(emb_hbm.at[idx_v.at[0]], rows_bufs[b],
                            sems[b]).wait()

    def process(i, rows):
      zero = jnp.zeros((16,), jnp.float32)

      def body(f, carry):
        s0, s1, s2, s3, q0, q1, q2, q3 = carry
        v0 = rows[f, pl.ds(0, 16)]
        v1 = rows[f, pl.ds(16, 16)]
        v2 = rows[f, pl.ds(32, 16)]
        v3 = rows[f, pl.ds(48, 16)]
        return (s0 + v0, s1 + v1, s2 + v2, s3 + v3,
                q0 + v0 * v0, q1 + v1 * v1, q2 + v2 * v2, q3 + v3 * v3)

      acc = lax.fori_loop(0, _FIELDS, body, (zero,) * 8, unroll=4)
      for c in range(4):
        s, q = acc[c], acc[4 + c]
        out_v[i, pl.ds(c * 16, 16)] = (s * s - q) * 0.5

    for b in range(nbuf):
      start(b, b)

    def step(j, carry):
      i0 = nbuf * j
      for b in range(nbuf):
        wait(b)
        process(i0 + b, rows_bufs[b])
        start(i0 + b + nbuf, b)
      return carry

    lax.fori_loop(0, _BPW // nbuf - 1, step, 0)
    i0 = _BPW - nbuf
    for b in range(nbuf):
      wait(b)
      process(i0 + b, rows_bufs[b])
    pltpu.sync_copy(out_v, h_hbm.at[pl.ds(base, _BPW)])

  return k(x_pad, emb)


def _mlp_tc(h, w1t, b1, w2t, b2, wf, bf):
  blk = 1024

  def body(h_ref, w1_ref, b1_ref, w2_ref, b2_ref, wf_ref, bf_ref, o_ref):
    hb = h_ref[...]
    a1 = jnp.maximum(
        jnp.dot(hb, w1_ref[...], preferred_element_type=jnp.float32)
        + b1_ref[...], 0.0)
    a2 = jnp.maximum(
        jnp.dot(a1, w2_ref[...], preferred_element_type=jnp.float32)
        + b2_ref[...], 0.0)
    t = jnp.sum(a2 * wf_ref[...], axis=1, keepdims=True) + bf_ref[...]
    o_ref[...] = 1.0 / (1.0 + jnp.exp(-t))

  return pl.pallas_call(
      body,
      grid=(_BATCH // blk,),
      in_specs=[
          pl.BlockSpec((blk, _DIM), lambda i: (i, 0)),
          pl.BlockSpec((_DIM, 32), lambda i: (0, 0)),
          pl.BlockSpec((1, 32), lambda i: (0, 0)),
          pl.BlockSpec((32, 16), lambda i: (0, 0)),
          pl.BlockSpec((1, 16), lambda i: (0, 0)),
          pl.BlockSpec((1, 16), lambda i: (0, 0)),
          pl.BlockSpec((1, 1), lambda i: (0, 0)),
      ],
      out_specs=pl.BlockSpec((blk, 1), lambda i: (i, 0)),
      out_shape=jax.ShapeDtypeStruct((_BATCH, 1), jnp.float32),
  )(h, w1t, b1, w2t, b2, wf, bf)


def kernel(x, Emb, W1, b1, W2, b2, Wf, bf):
  x = x.astype(jnp.int32)
  xp = jnp.pad(x, ((0, 0), (0, _FPAD - _FIELDS)))
  h = _bip_sc(xp, Emb)
  return _mlp_tc(h, W1.T, b1.reshape(1, -1), W2.T, b2.reshape(1, -1),
                 Wf, bf.reshape(1, 1))


# speedup vs baseline: 1.0021x; 1.0021x over previous
import functools

import jax
import jax.numpy as jnp
from jax import lax
from jax.experimental import pallas as pl
from jax.experimental.pallas import tpu as pltpu
from jax.experimental.pallas import tpu_sc as plsc

_BATCH = 16384
_FIELDS = 100
_FPAD = 104
_DIM = 64
_NC = 2
_NS = 16
_NW = _NC * _NS
_BPW = _BATCH // _NW
_W = 4
_NG = _BPW // _W


def _bip_sc(x_pad, emb):
  mesh = plsc.VectorSubcoreMesh(core_axis_name="c", subcore_axis_name="s")

  @functools.partial(
      pl.kernel,
      out_type=jax.ShapeDtypeStruct((_BATCH, _DIM), jnp.float32),
      mesh=mesh,
      scratch_types=(
          [pltpu.VMEM((_BPW * _FPAD,), jnp.int32)]
          + [pltpu.VMEM((_W * _FPAD, _DIM), jnp.float32)
             for _ in range(2)]
          + [pltpu.VMEM((_W, _DIM), jnp.float32)
             for _ in range(2)]
          + [pltpu.SemaphoreType.DMA for _ in range(4)]
      ),
      compiler_params=pltpu.CompilerParams(use_tc_tiling_on_sc=False),
  )
  def k(x_hbm, emb_hbm, h_hbm, idx_v, rows0, rows1, hb0, hb1,
        gsem0, gsem1, ssem0, ssem1):
    rows_bufs = (rows0, rows1)
    h_bufs = (hb0, hb1)
    gsems = (gsem0, gsem1)
    ssems = (ssem0, ssem1)
    wid = lax.axis_index("s") * _NC + lax.axis_index("c")
    base = wid * _BPW
    pltpu.sync_copy(x_hbm.at[wid], idx_v)

    def start_gather(g, b):
      pltpu.make_async_copy(
          emb_hbm.at[idx_v.at[pl.ds(g * (_W * _FPAD), _W * _FPAD)]],
          rows_bufs[b], gsems[b]).start()

    def wait_gather(b):
      pltpu.make_async_copy(
          emb_hbm.at[idx_v.at[pl.ds(0, _W * _FPAD)]],
          rows_bufs[b], gsems[b]).wait()

    def start_store(g, b):
      pltpu.make_async_copy(h_bufs[b], h_hbm.at[pl.ds(base + g * _W, _W)],
                            ssems[b]).start()

    def wait_store(b):
      pltpu.make_async_copy(h_bufs[b], h_hbm.at[pl.ds(base, _W)],
                            ssems[b]).wait()

    def process(b):
      rows = rows_bufs[b]
      zero = jnp.zeros((16,), jnp.float32)
      for s in range(_W):
        o = s * _FPAD

        def body(f, carry):
          s0, s1, s2, s3, q0, q1, q2, q3 = carry
          v0 = rows[o + f, pl.ds(0, 16)]
          v1 = rows[o + f, pl.ds(16, 16)]
          v2 = rows[o + f, pl.ds(32, 16)]
          v3 = rows[o + f, pl.ds(48, 16)]
          return (s0 + v0, s1 + v1, s2 + v2, s3 + v3,
                  q0 + v0 * v0, q1 + v1 * v1, q2 + v2 * v2, q3 + v3 * v3)

        acc = lax.fori_loop(0, _FIELDS, body, (zero,) * 8, unroll=4)
        for c in range(4):
          sm, q = acc[c], acc[4 + c]
          h_bufs[b][s, pl.ds(c * 16, 16)] = (sm * sm - q) * 0.5

    start_gather(0, 0)
    start_gather(1, 1)

    def step(j, carry):
      g0 = 2 * j
      for b in range(2):
        wait_gather(b)
        @pl.when(j > 0)
        def _():
          wait_store(b)
        process(b)
        start_store(g0 + b, b)
        start_gather(g0 + b + 2, b)
      return carry

    lax.fori_loop(0, _NG // 2 - 1, step, 0)
    for b in range(2):
      wait_gather(b)
      wait_store(b)
      process(b)
      start_store(_NG - 2 + b, b)
    for b in range(2):
      wait_store(b)

  return k(x_pad, emb)


def _mlp_tc(h, w1t, b1, w2t, b2, wf, bf):
  blk = 1024

  def body(h_ref, w1_ref, b1_ref, w2_ref, b2_ref, wf_ref, bf_ref, o_ref):
    hb = h_ref[...]
    a1 = jnp.maximum(
        jnp.dot(hb, w1_ref[...], preferred_element_type=jnp.float32)
        + b1_ref[...], 0.0)
    a2 = jnp.maximum(
        jnp.dot(a1, w2_ref[...], preferred_element_type=jnp.float32)
        + b2_ref[...], 0.0)
    t = jnp.sum(a2 * wf_ref[...], axis=1, keepdims=True) + bf_ref[...]
    o_ref[...] = 1.0 / (1.0 + jnp.exp(-t))

  return pl.pallas_call(
      body,
      grid=(_BATCH // blk,),
      in_specs=[
          pl.BlockSpec((blk, _DIM), lambda i: (i, 0)),
          pl.BlockSpec((_DIM, 32), lambda i: (0, 0)),
          pl.BlockSpec((1, 32), lambda i: (0, 0)),
          pl.BlockSpec((32, 16), lambda i: (0, 0)),
          pl.BlockSpec((1, 16), lambda i: (0, 0)),
          pl.BlockSpec((1, 16), lambda i: (0, 0)),
          pl.BlockSpec((1, 1), lambda i: (0, 0)),
      ],
      out_specs=pl.BlockSpec((blk, 1), lambda i: (i, 0)),
      out_shape=jax.ShapeDtypeStruct((_BATCH, 1), jnp.float32),
  )(h, w1t, b1, w2t, b2, wf, bf)


def kernel(x, Emb, W1, b1, W2, b2, Wf, bf):
  x = x.astype(jnp.int32)
  xp = jnp.pad(x, ((0, 0), (0, _FPAD - _FIELDS)))
  xw = xp.reshape(_NW, _BPW * _FPAD)
  h = _bip_sc(xw, Emb)
  return _mlp_tc(h, W1.T, b1.reshape(1, -1), W2.T, b2.reshape(1, -1),
                 Wf, bf.reshape(1, 1))
